# baseline (speedup 1.0000x reference)
import functools

import jax
import jax.numpy as jnp
from jax import lax
from jax.experimental import pallas as pl
from jax.experimental.pallas import tpu as pltpu
from jax.experimental.pallas import tpu_sc as plsc

B_TOK = 16384
DM = 768
NE = 64
KN = 8
NSEL = NE * KN


def _gelu_exact(a):
    return 0.5 * a * (1.0 + lax.erf(a * (2.0 ** -0.5)))


def _gate_body(x_ref, gw_ref, s_ref):
    s_ref[...] = jnp.dot(x_ref[...], gw_ref[...],
                         preferred_element_type=jnp.float32)


def _gating(x2, gw):
    blk = 2048
    return pl.pallas_call(
        _gate_body,
        grid=(B_TOK // blk,),
        in_specs=[
            pl.BlockSpec((blk, 2 * DM), lambda i: (i, 0)),
            pl.BlockSpec((2 * DM, NE), lambda i: (0, 0)),
        ],
        out_specs=pl.BlockSpec((blk, NE), lambda i: (i, 0)),
        out_shape=jax.ShapeDtypeStruct((B_TOK, NE), jnp.float32),
    )(x2, gw)


def _topk_body(s_ref, vals_ref, idx_ref):
    s = s_ref[...]
    rowid = lax.broadcasted_iota(jnp.int32, (B_TOK, NE), 0)
    for j in range(KN):
        m = jnp.max(s, axis=0, keepdims=True)
        cand = jnp.where(s == m, rowid, jnp.int32(2**31 - 1))
        am = jnp.min(cand, axis=0, keepdims=True)
        vals_ref[j:j + 1, :] = m
        idx_ref[j:j + 1, :] = am
        s = jnp.where(rowid == am, -jnp.inf, s)


def _topk(scores):
    return pl.pallas_call(
        _topk_body,
        out_shape=(
            jax.ShapeDtypeStruct((KN, NE), jnp.float32),
            jax.ShapeDtypeStruct((KN, NE), jnp.int32),
        ),
    )(scores)


def _sc_gather(x2, flat_idx):
    info = plsc.get_sparse_core_info()
    nw = info.num_cores * info.num_subcores
    bpw = NSEL // nw
    mesh = plsc.VectorSubcoreMesh(core_axis_name="c", subcore_axis_name="s")

    @functools.partial(
        pl.kernel,
        out_type=jax.ShapeDtypeStruct((NSEL, 2 * DM), jnp.float32),
        mesh=mesh,
        scratch_types=[
            pltpu.VMEM((bpw,), jnp.int32),
            pltpu.VMEM((bpw, 2 * DM), jnp.float32),
            pltpu.SemaphoreType.DMA,
        ],
    )
    def gk(x_hbm, idx_hbm, out_hbm, idx_v, rows_v, sem):
        wid = lax.axis_index("s") * info.num_cores + lax.axis_index("c")
        base = wid * bpw
        pltpu.sync_copy(idx_hbm.at[pl.ds(base, bpw)], idx_v)
        pltpu.async_copy(x_hbm.at[idx_v], rows_v, sem).wait()
        pltpu.sync_copy(rows_v, out_hbm.at[pl.ds(base, bpw)])

    return gk(x2, flat_idx)


def _expert_body(xg_ref, w_ref, sr_ref, si_ref, tv_ref, ywr_ref, ywi_ref):
    xg = xg_ref[...]
    sr = sr_ref[...]
    si = si_ref[...]
    xr = jnp.dot(xg, sr, preferred_element_type=jnp.float32)
    xi = jnp.dot(xg, si, preferred_element_type=jnp.float32)
    w2 = w_ref[0]
    a = jnp.dot(xr, w2, preferred_element_type=jnp.float32)
    b = jnp.dot(xi, w2, preferred_element_type=jnp.float32)
    yr = (jnp.dot(a, sr, preferred_element_type=jnp.float32)
          - jnp.dot(b, si, preferred_element_type=jnp.float32))
    yi = (jnp.dot(a, si, preferred_element_type=jnp.float32)
          + jnp.dot(b, sr, preferred_element_type=jnp.float32))
    v = tv_ref[...]
    ywr_ref[...] = yr * v
    ywi_ref[...] = yi * v


def _expert_matmuls(xg, ew3, sel_r, sel_i, tvals):
    return pl.pallas_call(
        _expert_body,
        grid=(NE,),
        in_specs=[
            pl.BlockSpec((KN, 2 * DM), lambda e: (e, 0)),
            pl.BlockSpec((1, DM, 2 * DM), lambda e: (e, 0, 0)),
            pl.BlockSpec((2 * DM, DM), lambda e: (0, 0)),
            pl.BlockSpec((2 * DM, DM), lambda e: (0, 0)),
            pl.BlockSpec((KN, 1), lambda e: (e, 0)),
        ],
        out_specs=(
            pl.BlockSpec((KN, DM), lambda e: (e, 0)),
            pl.BlockSpec((KN, DM), lambda e: (e, 0)),
        ),
        out_shape=(
            jax.ShapeDtypeStruct((NSEL, DM), jnp.float32),
            jax.ShapeDtypeStruct((NSEL, DM), jnp.float32),
        ),
    )(xg, ew3, sel_r, sel_i, tvals)


def _combine_body(ywr_ref, ywi_ref, fia_ref, fib_ref, er_ref, ei_ref,
                  bias_ref, rows_ref, cnt_ref):
    eq = (fia_ref[...] == fib_ref[...]).astype(jnp.float32)
    counts = jnp.sum(eq, axis=1, keepdims=True)
    sum_r = jnp.dot(eq, ywr_ref[...],
                    preferred_element_type=jnp.float32)
    sum_i = jnp.dot(eq, ywi_ref[...],
                    preferred_element_type=jnp.float32)
    avg_r = sum_r / counts
    avg_i = sum_i / counts
    ab = bias_ref[...]
    gr = _gelu_exact(avg_r + ab)
    gi = _gelu_exact(avg_i + ab)
    rows_ref[...] = (jnp.dot(gr, er_ref[...],
                             preferred_element_type=jnp.float32)
                     + jnp.dot(gi, ei_ref[...],
                               preferred_element_type=jnp.float32))
    cnt_ref[...] = counts


def _combine(ywr, ywi, fia, fib, exp_r, exp_i, bias_row):
    return pl.pallas_call(
        _combine_body,
        out_shape=(
            jax.ShapeDtypeStruct((NSEL, 2 * DM), jnp.float32),
            jax.ShapeDtypeStruct((NSEL, 1), jnp.float32),
        ),
    )(ywr, ywi, fia, fib, exp_r, exp_i, bias_row)


def _fill_body(bias2_ref, fib_ref, res_ref, cnt_ref, *, blk):
    i = pl.program_id(0)
    fill_row = _gelu_exact(bias2_ref[...])
    res_ref[...] = jnp.broadcast_to(fill_row, (blk, 2 * DM))
    tok = lax.broadcasted_iota(jnp.int32, (blk, NSEL), 0) + i * blk
    eq = (tok == fib_ref[...]).astype(jnp.float32)
    cnt_ref[...] = jnp.sum(eq, axis=1, keepdims=True)


def _fill(bias2, fib):
    blk = 2048
    return pl.pallas_call(
        functools.partial(_fill_body, blk=blk),
        grid=(B_TOK // blk,),
        in_specs=[
            pl.BlockSpec((1, 2 * DM), lambda i: (0, 0)),
            pl.BlockSpec((1, NSEL), lambda i: (0, 0)),
        ],
        out_specs=(
            pl.BlockSpec((blk, 2 * DM), lambda i: (i, 0)),
            pl.BlockSpec((blk, 1), lambda i: (i, 0)),
        ),
        out_shape=(
            jax.ShapeDtypeStruct((B_TOK, 2 * DM), jnp.float32),
            jax.ShapeDtypeStruct((B_TOK, 1), jnp.float32),
        ),
    )(bias2, fib)


def _scatter_body(sidx_ref, rows_ref, filled_ref, out_ref):
    del sidx_ref, filled_ref
    out_ref[...] = rows_ref[...]


def _scatter(flat_idx, rows, filled):
    grid_spec = pltpu.PrefetchScalarGridSpec(
        num_scalar_prefetch=1,
        grid=(NSEL,),
        in_specs=[
            pl.BlockSpec((1, 1, 2 * DM), lambda i, sidx: (i, 0, 0)),
            pl.BlockSpec((1, 1, 2 * DM), lambda i, sidx: (sidx[i], 0, 0)),
        ],
        out_specs=pl.BlockSpec((1, 1, 2 * DM), lambda i, sidx: (sidx[i], 0, 0)),
    )
    out3 = pl.pallas_call(
        _scatter_body,
        grid_spec=grid_spec,
        out_shape=jax.ShapeDtypeStruct((B_TOK, 1, 2 * DM), jnp.float32),
        input_output_aliases={2: 0},
    )(flat_idx, rows.reshape(NSEL, 1, 2 * DM),
      filled.reshape(B_TOK, 1, 2 * DM))
    return out3.reshape(B_TOK, 2 * DM)


def kernel(x, gate_weights, experts_weight, act_bias):
    x2 = x.reshape(B_TOK, 2 * DM)
    scores = _gating(x2, gate_weights)
    vals_t, idx_t = _topk(scores)

    flat_idx = idx_t.T.reshape(NSEL)
    xg = _sc_gather(x2, flat_idx)

    d_ar = lax.broadcasted_iota(jnp.int32, (2 * DM, DM), 0)
    d_ac = lax.broadcasted_iota(jnp.int32, (2 * DM, DM), 1)
    sel_r = (d_ar == 2 * d_ac).astype(jnp.float32)
    sel_i = (d_ar == 2 * d_ac + 1).astype(jnp.float32)
    exp_r = sel_r.T
    exp_i = sel_i.T

    ew3 = experts_weight.reshape(NE, DM, 2 * DM)
    vals_em = vals_t.T.reshape(NSEL, 1)
    ywr, ywi = _expert_matmuls(xg, ew3, sel_r, sel_i, vals_em)

    fia = flat_idx.reshape(NSEL, 1)
    fib = flat_idx.reshape(1, NSEL)
    bias_row = act_bias.reshape(1, DM)
    rows, _sel_counts = _combine(ywr, ywi, fia, fib, exp_r, exp_i, bias_row)

    bias2 = jnp.stack([act_bias, act_bias], axis=-1).reshape(1, 2 * DM)
    filled, cnt = _fill(bias2, fib)
    res2 = _scatter(flat_idx, rows, filled)

    res = res2.reshape(B_TOK, DM, 2)
    counts_buf = cnt.reshape(B_TOK, 1, 1)
    return res, idx_t, vals_t, counts_buf


# speedup vs baseline: 1.0091x; 1.0091x over previous
import functools

import jax
import jax.numpy as jnp
from jax import lax
from jax.experimental import pallas as pl
from jax.experimental.pallas import tpu as pltpu
from jax.experimental.pallas import tpu_sc as plsc

B_TOK = 16384
DM = 768
NE = 64
KN = 8
NSEL = NE * KN


def _gelu_exact(a):
    return 0.5 * a * (1.0 + lax.erf(a * (2.0 ** -0.5)))


def _gate_body(x_ref, gw_ref, s_ref):
    s_ref[...] = jnp.dot(x_ref[...], gw_ref[...],
                         preferred_element_type=jnp.float32)


def _gating(x2, gw):
    blk = 2048
    return pl.pallas_call(
        _gate_body,
        grid=(B_TOK // blk,),
        in_specs=[
            pl.BlockSpec((blk, 2 * DM), lambda i: (i, 0)),
            pl.BlockSpec((2 * DM, NE), lambda i: (0, 0)),
        ],
        out_specs=pl.BlockSpec((blk, NE), lambda i: (i, 0)),
        out_shape=jax.ShapeDtypeStruct((B_TOK, NE), jnp.float32),
    )(x2, gw)


def _topk_body(s_ref, vals_ref, idx_ref):
    s = s_ref[...]
    rowid = lax.broadcasted_iota(jnp.int32, (B_TOK, NE), 0)
    for j in range(KN):
        m = jnp.max(s, axis=0, keepdims=True)
        cand = jnp.where(s == m, rowid, jnp.int32(2**31 - 1))
        am = jnp.min(cand, axis=0, keepdims=True)
        vals_ref[j:j + 1, :] = m
        idx_ref[j:j + 1, :] = am
        s = jnp.where(rowid == am, -jnp.inf, s)


def _topk(scores):
    return pl.pallas_call(
        _topk_body,
        out_shape=(
            jax.ShapeDtypeStruct((KN, NE), jnp.float32),
            jax.ShapeDtypeStruct((KN, NE), jnp.int32),
        ),
    )(scores)


def _sc_gather(xp, flat_idx):
    info = plsc.get_sparse_core_info()
    nw = info.num_cores * info.num_subcores
    bpw = NSEL // nw
    mesh = plsc.VectorSubcoreMesh(core_axis_name="c", subcore_axis_name="s")

    @functools.partial(
        pl.kernel,
        out_type=jax.ShapeDtypeStruct((NSEL, 2 * DM), jnp.float32),
        mesh=mesh,
        scratch_types=[
            pltpu.VMEM((bpw,), jnp.int32),
            pltpu.VMEM((bpw, 2 * DM), jnp.float32),
            pltpu.SemaphoreType.DMA,
        ],
    )
    def gk(x_hbm, idx_hbm, out_hbm, idx_v, rows_v, sem):
        wid = lax.axis_index("s") * info.num_cores + lax.axis_index("c")
        base = wid * bpw
        pltpu.sync_copy(idx_hbm.at[pl.ds(base, bpw)], idx_v)
        pltpu.async_copy(x_hbm.at[idx_v], rows_v, sem).wait()
        pltpu.sync_copy(rows_v, out_hbm.at[pl.ds(base, bpw)])

    return gk(xp, flat_idx)


def _expert_body(xg_ref, w_ref, tv_ref, yw_ref):
    xg = xg_ref[...]
    xr = xg[:, :DM]
    xi = xg[:, DM:]
    w = w_ref[0]
    a = jnp.dot(xr, w, preferred_element_type=jnp.float32)
    b = jnp.dot(xi, w, preferred_element_type=jnp.float32)
    yr = a[:, :DM] - b[:, DM:]
    yi = a[:, DM:] + b[:, :DM]
    v = tv_ref[...]
    yw_ref[...] = jnp.concatenate([yr, yi], axis=1) * v


def _expert_matmuls(xg, ewp, tvals):
    return pl.pallas_call(
        _expert_body,
        grid=(NE,),
        in_specs=[
            pl.BlockSpec((KN, 2 * DM), lambda e: (e, 0)),
            pl.BlockSpec((1, DM, 2 * DM), lambda e: (e, 0, 0)),
            pl.BlockSpec((KN, 1), lambda e: (e, 0)),
        ],
        out_specs=pl.BlockSpec((KN, 2 * DM), lambda e: (e, 0)),
        out_shape=jax.ShapeDtypeStruct((NSEL, 2 * DM), jnp.float32),
    )(xg, ewp, tvals)


def _combine_body(yw_ref, fia_ref, fib_ref, bias2_ref, delta_ref):
    eq = (fia_ref[...] == fib_ref[...]).astype(jnp.float32)
    counts = jnp.sum(eq, axis=1, keepdims=True)
    summed = jnp.dot(eq, yw_ref[...],
                     preferred_element_type=jnp.float32)
    avg = summed / counts
    b2 = bias2_ref[...]
    fill_row = _gelu_exact(b2)
    delta_ref[...] = (_gelu_exact(avg + b2) - fill_row) / counts


def _combine(yw, fia, fib, bias2):
    return pl.pallas_call(
        _combine_body,
        out_shape=jax.ShapeDtypeStruct((NSEL, 2 * DM), jnp.float32),
    )(yw, fia, fib, bias2)


def _fill_body(bias2_ref, fib_ref, delta_ref, res_ref, cnt_ref, *, blk):
    i = pl.program_id(0)
    tok = lax.broadcasted_iota(jnp.int32, (blk, NSEL), 0) + i * blk
    oh = (tok == fib_ref[...]).astype(jnp.float32)
    cnt_ref[...] = jnp.sum(oh, axis=1, keepdims=True)
    fill_row = _gelu_exact(bias2_ref[...])
    res_ref[...] = fill_row + jnp.dot(oh, delta_ref[...],
                                      preferred_element_type=jnp.float32)


def _fill(bias2, fib, delta):
    blk = 2048
    return pl.pallas_call(
        functools.partial(_fill_body, blk=blk),
        grid=(B_TOK // blk,),
        in_specs=[
            pl.BlockSpec((1, 2 * DM), lambda i: (0, 0)),
            pl.BlockSpec((1, NSEL), lambda i: (0, 0)),
            pl.BlockSpec((NSEL, 2 * DM), lambda i: (0, 0)),
        ],
        out_specs=(
            pl.BlockSpec((blk, 2 * DM), lambda i: (i, 0)),
            pl.BlockSpec((blk, 1), lambda i: (i, 0)),
        ),
        out_shape=(
            jax.ShapeDtypeStruct((B_TOK, 2 * DM), jnp.float32),
            jax.ShapeDtypeStruct((B_TOK, 1), jnp.float32),
        ),
    )(bias2, fib, delta)


def kernel(x, gate_weights, experts_weight, act_bias):
    x2 = x.reshape(B_TOK, 2 * DM)
    scores = _gating(x2, gate_weights)
    vals_t, idx_t = _topk(scores)

    flat_idx = idx_t.T.reshape(NSEL)

    xp = jnp.transpose(x, (0, 2, 1)).reshape(B_TOK, 2 * DM)
    ewp = jnp.transpose(experts_weight, (0, 1, 3, 2)).reshape(NE, DM, 2 * DM)

    xg = _sc_gather(xp, flat_idx)

    vals_em = vals_t.T.reshape(NSEL, 1)
    yw = _expert_matmuls(xg, ewp, vals_em)

    fia = flat_idx.reshape(NSEL, 1)
    fib = flat_idx.reshape(1, NSEL)
    bias2 = jnp.concatenate([act_bias, act_bias]).reshape(1, 2 * DM)
    delta = _combine(yw, fia, fib, bias2)

    res_p, cnt = _fill(bias2, fib, delta)

    res = jnp.transpose(res_p.reshape(B_TOK, 2, DM), (0, 2, 1))
    counts_buf = cnt.reshape(B_TOK, 1, 1)
    return res, idx_t, vals_t, counts_buf


# speedup vs baseline: 1.4112x; 1.3984x over previous
import functools

import jax
import jax.numpy as jnp
from jax import lax
from jax.experimental import pallas as pl
from jax.experimental.pallas import tpu as pltpu
from jax.experimental.pallas import tpu_sc as plsc

B_TOK = 16384
DM = 768
NE = 64
KN = 8
NSEL = NE * KN


def _gelu_exact(a):
    return 0.5 * a * (1.0 + lax.erf(a * (2.0 ** -0.5)))


def _gate_body(x_ref, gw_ref, s_ref):
    s_ref[...] = jnp.dot(x_ref[...], gw_ref[...],
                         preferred_element_type=jnp.float32)


def _gating(x2, gw):
    blk = 2048
    return pl.pallas_call(
        _gate_body,
        grid=(B_TOK // blk,),
        in_specs=[
            pl.BlockSpec((blk, 2 * DM), lambda i: (i, 0)),
            pl.BlockSpec((2 * DM, NE), lambda i: (0, 0)),
        ],
        out_specs=pl.BlockSpec((blk, NE), lambda i: (i, 0)),
        out_shape=jax.ShapeDtypeStruct((B_TOK, NE), jnp.float32),
    )(x2, gw)


def _topk_body(s_ref, vals_ref, idx_ref):
    s = s_ref[...]
    rowid = lax.broadcasted_iota(jnp.int32, (B_TOK, NE), 0)
    for j in range(KN):
        m = jnp.max(s, axis=0, keepdims=True)
        cand = jnp.where(s == m, rowid, jnp.int32(2**31 - 1))
        am = jnp.min(cand, axis=0, keepdims=True)
        vals_ref[j:j + 1, :] = m
        idx_ref[j:j + 1, :] = am
        s = jnp.where(rowid == am, -jnp.inf, s)


def _topk(scores):
    return pl.pallas_call(
        _topk_body,
        out_shape=(
            jax.ShapeDtypeStruct((KN, NE), jnp.float32),
            jax.ShapeDtypeStruct((KN, NE), jnp.int32),
        ),
    )(scores)


def _sc_gather(xp, flat_idx):
    info = plsc.get_sparse_core_info()
    nw = info.num_cores * info.num_subcores
    bpw = NSEL // nw
    mesh = plsc.VectorSubcoreMesh(core_axis_name="c", subcore_axis_name="s")

    @functools.partial(
        pl.kernel,
        out_type=jax.ShapeDtypeStruct((NSEL, 2 * DM), jnp.float32),
        mesh=mesh,
        scratch_types=[
            pltpu.VMEM((bpw,), jnp.int32),
            pltpu.VMEM((bpw, 2 * DM), jnp.float32),
            pltpu.SemaphoreType.DMA,
        ],
    )
    def gk(x_hbm, idx_hbm, out_hbm, idx_v, rows_v, sem):
        wid = lax.axis_index("s") * info.num_cores + lax.axis_index("c")
        base = wid * bpw
        pltpu.sync_copy(idx_hbm.at[pl.ds(base, bpw)], idx_v)
        pltpu.async_copy(x_hbm.at[idx_v], rows_v, sem).wait()
        pltpu.sync_copy(rows_v, out_hbm.at[pl.ds(base, bpw)])

    return gk(xp, flat_idx)


def _expert_body(xg_ref, w_ref, tv_ref, sr_ref, si_ref, yw_ref):
    xg = xg_ref[...]
    xr = jnp.dot(xg, sr_ref[...], preferred_element_type=jnp.float32)
    xi = jnp.dot(xg, si_ref[...], preferred_element_type=jnp.float32)
    w2 = w_ref[0]
    a = jnp.dot(xr, w2, preferred_element_type=jnp.float32)
    b = jnp.dot(xi, w2, preferred_element_type=jnp.float32)
    colpar = lax.broadcasted_iota(jnp.int32, (KN, 2 * DM), 1) % 2
    c = jnp.where(colpar == 0,
                  -jnp.roll(b, -1, axis=1),
                  jnp.roll(b, 1, axis=1))
    yw_ref[...] = (a + c) * tv_ref[...]


def _expert_matmuls(xg, w2, tvals, sr, si):
    return pl.pallas_call(
        _expert_body,
        grid=(NE,),
        in_specs=[
            pl.BlockSpec((KN, 2 * DM), lambda e: (e, 0)),
            pl.BlockSpec((1, DM, 2 * DM), lambda e: (e, 0, 0)),
            pl.BlockSpec((KN, 1), lambda e: (e, 0)),
            pl.BlockSpec((2 * DM, DM), lambda e: (0, 0)),
            pl.BlockSpec((2 * DM, DM), lambda e: (0, 0)),
        ],
        out_specs=pl.BlockSpec((KN, 2 * DM), lambda e: (e, 0)),
        out_shape=jax.ShapeDtypeStruct((NSEL, 2 * DM), jnp.float32),
    )(xg, w2, tvals, sr, si)


def _combine_body(yw_ref, fia_ref, fib_ref, bias2_ref, delta_ref):
    eq = (fia_ref[...] == fib_ref[...]).astype(jnp.float32)
    counts = jnp.sum(eq, axis=1, keepdims=True)
    summed = jnp.dot(eq, yw_ref[...],
                     preferred_element_type=jnp.float32)
    avg = summed / counts
    b2 = bias2_ref[...]
    fill_row = _gelu_exact(b2)
    delta_ref[...] = (_gelu_exact(avg + b2) - fill_row) / counts


def _combine(yw, fia, fib, bias2):
    return pl.pallas_call(
        _combine_body,
        out_shape=jax.ShapeDtypeStruct((NSEL, 2 * DM), jnp.float32),
    )(yw, fia, fib, bias2)


def _fill_body(bias2_ref, fib_ref, delta_ref, res_ref, cnt_ref, *, blk):
    i = pl.program_id(0)
    tok = lax.broadcasted_iota(jnp.int32, (blk, NSEL), 0) + i * blk
    oh = (tok == fib_ref[...]).astype(jnp.float32)
    cnt_ref[...] = jnp.sum(oh, axis=1, keepdims=True)
    fill_row = _gelu_exact(bias2_ref[...])
    res_ref[...] = fill_row + jnp.dot(oh, delta_ref[...],
                                      preferred_element_type=jnp.float32)


def _fill(bias2, fib, delta):
    blk = 2048
    return pl.pallas_call(
        functools.partial(_fill_body, blk=blk),
        grid=(B_TOK // blk,),
        in_specs=[
            pl.BlockSpec((1, 2 * DM), lambda i: (0, 0)),
            pl.BlockSpec((1, NSEL), lambda i: (0, 0)),
            pl.BlockSpec((NSEL, 2 * DM), lambda i: (0, 0)),
        ],
        out_specs=(
            pl.BlockSpec((blk, 2 * DM), lambda i: (i, 0)),
            pl.BlockSpec((blk, 1), lambda i: (i, 0)),
        ),
        out_shape=(
            jax.ShapeDtypeStruct((B_TOK, 2 * DM), jnp.float32),
            jax.ShapeDtypeStruct((B_TOK, 1), jnp.float32),
        ),
    )(bias2, fib, delta)


def kernel(x, gate_weights, experts_weight, act_bias):
    x2 = x.reshape(B_TOK, 2 * DM)
    scores = _gating(x2, gate_weights)
    vals_t, idx_t = _topk(scores)

    flat_idx = idx_t.T.reshape(NSEL)

    xg = _sc_gather(x2, flat_idx)

    w2 = experts_weight.reshape(NE, DM, 2 * DM)

    av = jnp.arange(2 * DM, dtype=jnp.int32)[:, None]
    jv = jnp.arange(DM, dtype=jnp.int32)[None, :]
    sr = (av == 2 * jv).astype(jnp.float32)
    si = (av == 2 * jv + 1).astype(jnp.float32)

    vals_em = vals_t.T.reshape(NSEL, 1)
    yw = _expert_matmuls(xg, w2, vals_em, sr, si)

    fia = flat_idx.reshape(NSEL, 1)
    fib = flat_idx.reshape(1, NSEL)
    bias2 = jnp.repeat(act_bias, 2).reshape(1, 2 * DM)
    delta = _combine(yw, fia, fib, bias2)

    res_p, cnt = _fill(bias2, fib, delta)

    res = res_p.reshape(B_TOK, DM, 2)
    counts_buf = cnt.reshape(B_TOK, 1, 1)
    return res, idx_t, vals_t, counts_buf
